# baseline (speedup 1.0000x reference)
import functools

import jax
import jax.numpy as jnp
from jax import lax
from jax.experimental import pallas as pl
from jax.experimental.pallas import tpu as pltpu
from jax.experimental.pallas import tpu_sc as plsc

N = 10000
E = 320000
D = 128
DH = D // 2
C = 32

L = 16
NC = 2
NS = 16
CHUNK = 128

EDGES_TILE = 20480
ROWS = EDGES_TILE // CHUNK
PAD_TILE = EDGES_TILE - E // NS
QS = DH // L
N_DEN = 10016
N_OUT = 10008
OUT_ROWS_TILE = 624
OUT_ROWS_REM = N - NS * OUT_ROWS_TILE
OUT_REM_BASE = NS * OUT_ROWS_TILE
OUT_ZERO_REM = N_OUT - NS * OUT_ROWS_TILE

_NEG_SLOPE = 0.2


def _dense_common(x, W, a_s, a_d, hs_ref, als_ref, ald_ref):
    h = jnp.dot(x.astype(jnp.bfloat16), W.astype(jnp.bfloat16),
                preferred_element_type=jnp.float32)
    hs_ref[0] = h[:, :DH]
    hs_ref[1] = h[:, DH:]
    h16 = h.astype(jnp.bfloat16)
    als_ref[...] = jnp.dot(h16, a_s.astype(jnp.bfloat16),
                           preferred_element_type=jnp.float32)
    ald_ref[...] = jnp.dot(h16, a_d.astype(jnp.bfloat16),
                           preferred_element_type=jnp.float32)


def _dense1_body(x_ref, w_ref, as_ref, ad_ref, hs_ref, als_ref, ald_ref):
    _dense_common(x_ref[...], w_ref[...], as_ref[...], ad_ref[...],
                  hs_ref, als_ref, ald_ref)


def _dense2_body(o_ref, w_ref, as_ref, ad_ref, hs_ref, als_ref, ald_ref):
    t = jnp.maximum(jnp.concatenate([o_ref[0], o_ref[1]], axis=-1), 0.0)
    _dense_common(t, w_ref[...], as_ref[...], ad_ref[...],
                  hs_ref, als_ref, ald_ref)


_dense_out = [
    jax.ShapeDtypeStruct((NC, N, DH), jnp.float32),
    jax.ShapeDtypeStruct((N,), jnp.float32),
    jax.ShapeDtypeStruct((N,), jnp.float32),
]

_dense1 = pl.pallas_call(_dense1_body, out_shape=_dense_out)
_dense2 = pl.pallas_call(_dense2_body, out_shape=_dense_out)


_mesh = plsc.VectorSubcoreMesh(core_axis_name="c", subcore_axis_name="s")
_sc_params = pltpu.CompilerParams(needs_layout_passes=False,
                                  use_tc_tiling_on_sc=False)


@functools.partial(
    pl.kernel,
    out_type=jax.ShapeDtypeStruct((NC, N, DH), jnp.float32),
    mesh=_mesh,
    compiler_params=_sc_params,
    scratch_types=[
        pltpu.VMEM((N,), jnp.float32),
        pltpu.VMEM((N_DEN,), jnp.float32),
        pltpu.VMEM((N_DEN,), jnp.float32),
        pltpu.VMEM((EDGES_TILE,), jnp.int32),
        pltpu.VMEM((ROWS, CHUNK), jnp.int32),
        pltpu.VMEM((CHUNK, DH), jnp.float32),
        pltpu.VMEM((CHUNK, DH), jnp.float32),
        pltpu.VMEM((5, CHUNK), jnp.float32),
        pltpu.VMEM((CHUNK,), jnp.float32),
        pltpu.VMEM_SHARED((N_DEN,), jnp.float32),
        pltpu.VMEM_SHARED((N_OUT, DH), jnp.float32),
        pltpu.SemaphoreType.DMA,
        pltpu.SemaphoreType.DMA,
        pltpu.SemaphoreType.DMA,
        pltpu.SemaphoreType.DMA,
        pltpu.SemaphoreType.DMA,
    ],
)
def _edge_kernel(as_hbm, ad_hbm, h_hbm, src_hbm, dst_p1_hbm,
                 out_hbm, asv, adv, denv, srcv, dstv2, rowsa, rowsb,
                 pbufs, pbuf, den_sh, out_sh, s1, ga, gb, sa, sb):
    cid = lax.axis_index("c")
    sid = lax.axis_index("s")
    zero16 = jnp.zeros((L,), jnp.float32)

    def _zrow(r, carry):
        for q in range(QS):
            rowsa[r, pl.ds(q * L, L)] = zero16
        return carry
    lax.fori_loop(0, CHUNK, _zrow, 0)

    def _zden(i, carry):
        denv[pl.ds(i * L, L)] = zero16
        return carry
    lax.fori_loop(0, N_DEN // L, _zden, 0)

    base = sid * OUT_ROWS_TILE
    nfull = OUT_ROWS_TILE // CHUNK
    rem = OUT_ROWS_TILE - nfull * CHUNK
    for k in range(nfull):
        pltpu.sync_copy(rowsa, out_sh.at[pl.ds(base + k * CHUNK, CHUNK)])
    pltpu.sync_copy(rowsa.at[pl.ds(0, rem)],
                    out_sh.at[pl.ds(base + nfull * CHUNK, rem)])

    @pl.when(sid == 0)
    def _():
        pltpu.sync_copy(rowsa.at[pl.ds(0, OUT_ZERO_REM)],
                        out_sh.at[pl.ds(OUT_REM_BASE, OUT_ZERO_REM)])
        pltpu.sync_copy(denv, den_sh)

    plsc.subcore_barrier()

    pltpu.sync_copy(as_hbm, asv)
    pltpu.sync_copy(ad_hbm, adv.at[pl.ds(0, N)])
    adv[pl.ds(N, N_DEN - N)] = jnp.zeros((N_DEN - N,), jnp.float32)
    pltpu.sync_copy(src_hbm.at[pl.ds(sid * EDGES_TILE, EDGES_TILE)], srcv)
    pltpu.sync_copy(dst_p1_hbm.at[sid], dstv2)

    def _edge_p(s16, d16):
        a1 = plsc.load_gather(asv, [s16])
        a2 = plsc.load_gather(adv, [d16])
        e = a1 + a2
        w = jnp.where(e >= 0.0, e, e * _NEG_SLOPE)
        return jnp.exp(w)

    def _p1(t, carry):
        handles = []
        for j in range(5):
            r = t * 5 + j
            for g in range(CHUNK // L):
                s16 = srcv[pl.ds(r * CHUNK + g * L, L)]
                d16 = dstv2[r, pl.ds(g * L, L)]
                pbufs[j, pl.ds(g * L, L)] = _edge_p(s16, d16)
            handles.append(pltpu.async_copy(
                pbufs.at[j], den_sh.at[dstv2.at[r]], s1, add=True))
        for hnd in handles:
            hnd.wait()
        return carry
    lax.fori_loop(0, ROWS // 5, _p1, 0)

    plsc.subcore_barrier()

    pltpu.sync_copy(den_sh, denv)

    def _issue_gather(r, buf, sem):
        pltpu.async_copy(
            h_hbm.at[cid].at[srcv.at[pl.ds(r * CHUNK, CHUNK)]], buf, sem)

    def _drain(buf, sem):
        pltpu.make_async_copy(h_hbm.at[0].at[pl.ds(0, CHUNK)], buf, sem).wait()

    def _coef_scale(r, rows):
        for g in range(CHUNK // L):
            s16 = srcv[pl.ds(r * CHUNK + g * L, L)]
            d16 = dstv2[r, pl.ds(g * L, L)]
            p = _edge_p(s16, d16)
            dv = plsc.load_gather(denv, [d16])
            pbuf[pl.ds(g * L, L)] = p / (dv + 1e-16)

        def _scale(jj, c2):
            j0 = jj * 8
            for u in range(8):
                j = j0 + u
                c16 = plsc.load_gather(pbuf, [jnp.full((L,), j, jnp.int32)])
                for q in range(QS):
                    rows[j, pl.ds(q * L, L)] = rows[j, pl.ds(q * L, L)] * c16
            return c2
        lax.fori_loop(0, CHUNK // 8, _scale, 0)

    _issue_gather(0, rowsa, ga)
    _issue_gather(1, rowsb, gb)

    def _p2(k, carry):
        r0 = 2 * k
        r1 = 2 * k + 1
        _drain(rowsa, ga)
        _coef_scale(r0, rowsa)
        pltpu.async_copy(rowsa, out_sh.at[dstv2.at[r0]], sa, add=True)
        _drain(rowsb, gb)
        _coef_scale(r1, rowsb)
        pltpu.async_copy(rowsb, out_sh.at[dstv2.at[r1]], sb, add=True)
        _drain(rowsa, sa)
        _issue_gather(jnp.minimum(r0 + 2, ROWS - 1), rowsa, ga)
        _drain(rowsb, sb)
        _issue_gather(jnp.minimum(r1 + 2, ROWS - 1), rowsb, gb)
        return carry
    lax.fori_loop(0, ROWS // 2, _p2, 0)

    _drain(rowsa, ga)
    _drain(rowsb, gb)

    plsc.subcore_barrier()

    pltpu.sync_copy(out_sh.at[pl.ds(base, OUT_ROWS_TILE)],
                    out_hbm.at[cid].at[pl.ds(base, OUT_ROWS_TILE)])

    @pl.when(sid == 0)
    def _():
        pltpu.sync_copy(out_sh.at[pl.ds(OUT_REM_BASE, OUT_ROWS_REM)],
                        out_hbm.at[cid].at[pl.ds(OUT_REM_BASE, OUT_ROWS_REM)])


@functools.partial(
    pl.kernel,
    out_type=jax.ShapeDtypeStruct((C,), jnp.float32),
    mesh=_mesh,
    compiler_params=_sc_params,
    scratch_types=[
        pltpu.VMEM((C,), jnp.int32),
        pltpu.VMEM((C, DH), jnp.float32),
        pltpu.VMEM((C, DH), jnp.float32),
        pltpu.VMEM((D,), jnp.float32),
        pltpu.VMEM((C,), jnp.float32),
    ],
)
def _score_kernel(o_hbm, cand_hbm, w_hbm, out_hbm, candv, r0, r1, wv, sv):
    cid = lax.axis_index("c")
    sid = lax.axis_index("s")

    @pl.when((cid == 0) & (sid == 0))
    def _():
        pltpu.sync_copy(cand_hbm, candv)
        pltpu.sync_copy(w_hbm, wv)
        pltpu.sync_copy(o_hbm.at[0].at[candv], r0)
        pltpu.sync_copy(o_hbm.at[1].at[candv], r1)

        def _b16(v):
            u = plsc.bitcast(v, jnp.int32)
            r = u + 0x7FFF + ((u >> 16) & 1)
            return plsc.bitcast(r & jnp.int32(-65536), jnp.float32)

        for g in range(C // L):
            rid = lax.iota(jnp.int32, L) + g * L

            def _f(f, acc):
                f16 = jnp.full((L,), f, jnp.int32)
                c0 = _b16(jnp.maximum(plsc.load_gather(r0, [rid, f16]), 0.0))
                c1 = _b16(jnp.maximum(plsc.load_gather(r1, [rid, f16]), 0.0))
                w0 = _b16(plsc.load_gather(wv, [f16]))
                w1 = _b16(plsc.load_gather(wv, [f16 + DH]))
                return acc + c0 * w0 + c1 * w1
            acc = lax.fori_loop(0, DH, _f, jnp.zeros((L,), jnp.float32))
            sv[pl.ds(g * L, L)] = acc
        pltpu.sync_copy(sv, out_hbm)


def kernel(x, edge_index, cand_idxs, W1, a_src1, a_dst1,
           W2, a_src2, a_dst2, w_score, b_score):
    src2 = edge_index[0].reshape(NS, E // NS)
    dst2 = edge_index[1].reshape(NS, E // NS)
    pad_s = jnp.zeros((NS, PAD_TILE), jnp.int32)
    pad_d = jnp.full((NS, PAD_TILE), N, jnp.int32)
    src = jnp.concatenate([src2, pad_s], axis=1).reshape(-1)
    dst_p1 = jnp.concatenate([dst2, pad_d], axis=1).reshape(NS, ROWS, CHUNK)

    hs1, as1, ad1 = _dense1(x, W1, a_src1, a_dst1)
    out1 = _edge_kernel(as1, ad1, hs1, src, dst_p1)
    hs2, as2, ad2 = _dense2(out1, W2, a_src2, a_dst2)
    out2 = _edge_kernel(as2, ad2, hs2, src, dst_p1)
    scores = _score_kernel(out2, cand_idxs, w_score)
    return scores + b_score


# speedup vs baseline: 1.5436x; 1.5436x over previous
import functools

import jax
import jax.numpy as jnp
from jax import lax
from jax.experimental import pallas as pl
from jax.experimental.pallas import tpu as pltpu
from jax.experimental.pallas import tpu_sc as plsc

N = 10000
E = 320000
D = 128
DH = D // 2
C = 32

L = 16
NC = 2
NS = 16
CHUNK = 80

EDGES_TILE = E // NS
ROWS = EDGES_TILE // CHUNK
QS = DH // L
N_DEN = N
N_OUT = N
OUT_ROWS_TILE = 624
OUT_ROWS_REM = N - NS * OUT_ROWS_TILE
OUT_REM_BASE = NS * OUT_ROWS_TILE
OUT_ZERO_REM = N_OUT - NS * OUT_ROWS_TILE

_NEG_SLOPE = 0.2


def _dense_common(x, W, a_s, a_d, hs_ref, als_ref, ald_ref):
    h = jnp.dot(x.astype(jnp.bfloat16), W.astype(jnp.bfloat16),
                preferred_element_type=jnp.float32)
    hs_ref[0] = h[:, :DH]
    hs_ref[1] = h[:, DH:]
    h16 = h.astype(jnp.bfloat16)
    als_ref[...] = jnp.dot(h16, a_s.astype(jnp.bfloat16),
                           preferred_element_type=jnp.float32)
    ald_ref[...] = jnp.dot(h16, a_d.astype(jnp.bfloat16),
                           preferred_element_type=jnp.float32)


def _dense1_body(x_ref, w_ref, as_ref, ad_ref, hs_ref, als_ref, ald_ref):
    _dense_common(x_ref[...], w_ref[...], as_ref[...], ad_ref[...],
                  hs_ref, als_ref, ald_ref)


def _dense2_body(o_ref, w_ref, as_ref, ad_ref, hs_ref, als_ref, ald_ref):
    t = jnp.maximum(jnp.concatenate([o_ref[0], o_ref[1]], axis=-1), 0.0)
    _dense_common(t, w_ref[...], as_ref[...], ad_ref[...],
                  hs_ref, als_ref, ald_ref)


_dense_out = [
    jax.ShapeDtypeStruct((NC, N, DH), jnp.float32),
    jax.ShapeDtypeStruct((N,), jnp.float32),
    jax.ShapeDtypeStruct((N,), jnp.float32),
]

_dense1 = pl.pallas_call(_dense1_body, out_shape=_dense_out)
_dense2 = pl.pallas_call(_dense2_body, out_shape=_dense_out)


_mesh = plsc.VectorSubcoreMesh(core_axis_name="c", subcore_axis_name="s")
_sc_params = pltpu.CompilerParams(needs_layout_passes=False,
                                  use_tc_tiling_on_sc=False)


@functools.partial(
    pl.kernel,
    out_type=jax.ShapeDtypeStruct((NC, N, DH), jnp.float32),
    mesh=_mesh,
    compiler_params=_sc_params,
    scratch_types=[
        pltpu.VMEM((N,), jnp.float32),
        pltpu.VMEM((N_DEN,), jnp.float32),
        pltpu.VMEM((N_DEN,), jnp.float32),
        pltpu.VMEM((EDGES_TILE,), jnp.int32),
        pltpu.VMEM((ROWS, CHUNK), jnp.int32),
        pltpu.VMEM((CHUNK, DH), jnp.float32),
        pltpu.VMEM((CHUNK, DH), jnp.float32),
        pltpu.VMEM((5, CHUNK), jnp.float32),
        pltpu.VMEM((CHUNK,), jnp.float32),
        pltpu.VMEM_SHARED((N_DEN,), jnp.float32),
        pltpu.VMEM_SHARED((N_OUT, DH), jnp.float32),
        pltpu.SemaphoreType.DMA,
        pltpu.SemaphoreType.DMA,
        pltpu.SemaphoreType.DMA,
        pltpu.SemaphoreType.DMA,
        pltpu.SemaphoreType.DMA,
    ],
)
def _edge_kernel(as_hbm, ad_hbm, h_hbm, src_hbm, dst_p1_hbm,
                 out_hbm, asv, adv, denv, srcv, dstv2, rowsa, rowsb,
                 pbufs, pbuf, den_sh, out_sh, s1, ga, gb, sa, sb):
    cid = lax.axis_index("c")
    sid = lax.axis_index("s")
    zero16 = jnp.zeros((L,), jnp.float32)

    def _zrow(r, carry):
        for q in range(QS):
            rowsa[r, pl.ds(q * L, L)] = zero16
        return carry
    lax.fori_loop(0, CHUNK, _zrow, 0)

    def _zden(i, carry):
        denv[pl.ds(i * L, L)] = zero16
        return carry
    lax.fori_loop(0, N_DEN // L, _zden, 0)

    base = sid * OUT_ROWS_TILE
    nfull = OUT_ROWS_TILE // CHUNK
    rem = OUT_ROWS_TILE - nfull * CHUNK
    for k in range(nfull):
        pltpu.sync_copy(rowsa, out_sh.at[pl.ds(base + k * CHUNK, CHUNK)])
    pltpu.sync_copy(rowsa.at[pl.ds(0, rem)],
                    out_sh.at[pl.ds(base + nfull * CHUNK, rem)])

    @pl.when(sid == 0)
    def _():
        pltpu.sync_copy(rowsa.at[pl.ds(0, OUT_ZERO_REM)],
                        out_sh.at[pl.ds(OUT_REM_BASE, OUT_ZERO_REM)])
        pltpu.sync_copy(denv, den_sh)

    plsc.subcore_barrier()

    pltpu.sync_copy(as_hbm, asv)
    pltpu.sync_copy(ad_hbm, adv)
    pltpu.sync_copy(src_hbm.at[pl.ds(sid * EDGES_TILE, EDGES_TILE)], srcv)
    pltpu.sync_copy(dst_p1_hbm.at[sid], dstv2)

    def _edge_p(s16, d16):
        a1 = plsc.load_gather(asv, [s16])
        a2 = plsc.load_gather(adv, [d16])
        e = a1 + a2
        w = jnp.where(e >= 0.0, e, e * _NEG_SLOPE)
        return jnp.exp(w)

    def _p1(t, carry):
        handles = []
        for j in range(5):
            r = t * 5 + j
            for g in range(CHUNK // L):
                s16 = srcv[pl.ds(r * CHUNK + g * L, L)]
                d16 = dstv2[r, pl.ds(g * L, L)]
                pbufs[j, pl.ds(g * L, L)] = _edge_p(s16, d16)
            handles.append(pltpu.async_copy(
                pbufs.at[j], den_sh.at[dstv2.at[r]], s1, add=True))
        for hnd in handles:
            hnd.wait()
        return carry
    lax.fori_loop(0, ROWS // 5, _p1, 0)

    plsc.subcore_barrier()

    pltpu.sync_copy(den_sh, denv)

    def _issue_gather(r, buf, sem):
        pltpu.async_copy(
            h_hbm.at[cid].at[srcv.at[pl.ds(r * CHUNK, CHUNK)]], buf, sem)

    def _drain(buf, sem):
        pltpu.make_async_copy(h_hbm.at[0].at[pl.ds(0, CHUNK)], buf, sem).wait()

    def _coef_scale(r, rows):
        for g in range(CHUNK // L):
            s16 = srcv[pl.ds(r * CHUNK + g * L, L)]
            d16 = dstv2[r, pl.ds(g * L, L)]
            p = _edge_p(s16, d16)
            dv = plsc.load_gather(denv, [d16])
            pbuf[pl.ds(g * L, L)] = p / (dv + 1e-16)

        def _scale(jj, c2):
            j0 = jj * 8
            for u in range(8):
                j = j0 + u
                c16 = plsc.load_gather(pbuf, [jnp.full((L,), j, jnp.int32)])
                for q in range(QS):
                    rows[j, pl.ds(q * L, L)] = rows[j, pl.ds(q * L, L)] * c16
            return c2
        lax.fori_loop(0, CHUNK // 8, _scale, 0)

    _issue_gather(0, rowsa, ga)
    _issue_gather(1, rowsb, gb)

    def _p2(k, carry):
        r0 = 2 * k
        r1 = 2 * k + 1
        _drain(rowsa, ga)
        _coef_scale(r0, rowsa)
        pltpu.async_copy(rowsa, out_sh.at[dstv2.at[r0]], sa, add=True)
        _drain(rowsb, gb)
        _coef_scale(r1, rowsb)
        pltpu.async_copy(rowsb, out_sh.at[dstv2.at[r1]], sb, add=True)
        _drain(rowsa, sa)
        _issue_gather(jnp.minimum(r0 + 2, ROWS - 1), rowsa, ga)
        _drain(rowsb, sb)
        _issue_gather(jnp.minimum(r1 + 2, ROWS - 1), rowsb, gb)
        return carry
    lax.fori_loop(0, ROWS // 2, _p2, 0)

    _drain(rowsa, ga)
    _drain(rowsb, gb)

    plsc.subcore_barrier()

    pltpu.sync_copy(out_sh.at[pl.ds(base, OUT_ROWS_TILE)],
                    out_hbm.at[cid].at[pl.ds(base, OUT_ROWS_TILE)])

    @pl.when(sid == 0)
    def _():
        pltpu.sync_copy(out_sh.at[pl.ds(OUT_REM_BASE, OUT_ROWS_REM)],
                        out_hbm.at[cid].at[pl.ds(OUT_REM_BASE, OUT_ROWS_REM)])


@functools.partial(
    pl.kernel,
    out_type=jax.ShapeDtypeStruct((C,), jnp.float32),
    mesh=_mesh,
    compiler_params=_sc_params,
    scratch_types=[
        pltpu.VMEM((C,), jnp.int32),
        pltpu.VMEM((C, DH), jnp.float32),
        pltpu.VMEM((C, DH), jnp.float32),
        pltpu.VMEM((D,), jnp.float32),
        pltpu.VMEM((C,), jnp.float32),
    ],
)
def _score_kernel(o_hbm, cand_hbm, w_hbm, out_hbm, candv, r0, r1, wv, sv):
    cid = lax.axis_index("c")
    sid = lax.axis_index("s")

    @pl.when((cid == 0) & (sid == 0))
    def _():
        pltpu.sync_copy(cand_hbm, candv)
        pltpu.sync_copy(w_hbm, wv)
        pltpu.sync_copy(o_hbm.at[0].at[candv], r0)
        pltpu.sync_copy(o_hbm.at[1].at[candv], r1)

        def _b16(v):
            u = plsc.bitcast(v, jnp.int32)
            r = u + 0x7FFF + ((u >> 16) & 1)
            return plsc.bitcast(r & jnp.int32(-65536), jnp.float32)

        for g in range(C // L):
            rid = lax.iota(jnp.int32, L) + g * L

            def _f(f, acc):
                f16 = jnp.full((L,), f, jnp.int32)
                c0 = _b16(jnp.maximum(plsc.load_gather(r0, [rid, f16]), 0.0))
                c1 = _b16(jnp.maximum(plsc.load_gather(r1, [rid, f16]), 0.0))
                w0 = _b16(plsc.load_gather(wv, [f16]))
                w1 = _b16(plsc.load_gather(wv, [f16 + DH]))
                return acc + c0 * w0 + c1 * w1
            acc = lax.fori_loop(0, DH, _f, jnp.zeros((L,), jnp.float32))
            sv[pl.ds(g * L, L)] = acc
        pltpu.sync_copy(sv, out_hbm)


def kernel(x, edge_index, cand_idxs, W1, a_src1, a_dst1,
           W2, a_src2, a_dst2, w_score, b_score):
    src = edge_index[0]
    dst_p1 = edge_index[1].reshape(NS, ROWS, CHUNK)

    hs1, as1, ad1 = _dense1(x, W1, a_src1, a_dst1)
    out1 = _edge_kernel(as1, ad1, hs1, src, dst_p1)
    hs2, as2, ad2 = _dense2(out1, W2, a_src2, a_dst2)
    out2 = _edge_kernel(as2, ad2, hs2, src, dst_p1)
    scores = _score_kernel(out2, cand_idxs, w_score)
    return scores + b_score


# speedup vs baseline: 1.6484x; 1.0679x over previous
import functools

import jax
import jax.numpy as jnp
from jax import lax
from jax.experimental import pallas as pl
from jax.experimental.pallas import tpu as pltpu
from jax.experimental.pallas import tpu_sc as plsc

N = 10000
E = 320000
D = 128
DH = D // 2
C = 32

L = 16
NC = 2
NS = 16
CHUNK = 80

EDGES_TILE = E // NS
ROWS = EDGES_TILE // CHUNK
QS = DH // L
N_DEN = N
N_OUT = N
OUT_ROWS_TILE = 624
OUT_ROWS_REM = N - NS * OUT_ROWS_TILE
OUT_REM_BASE = NS * OUT_ROWS_TILE
OUT_ZERO_REM = N_OUT - NS * OUT_ROWS_TILE

_NEG_SLOPE = 0.2


def _dense_common(x, W, a_s, a_d, hs_ref, als_ref, ald_ref):
    h = jnp.dot(x.astype(jnp.bfloat16), W.astype(jnp.bfloat16),
                preferred_element_type=jnp.float32)
    hs_ref[0] = h[:, :DH]
    hs_ref[1] = h[:, DH:]
    h16 = h.astype(jnp.bfloat16)
    als_ref[...] = jnp.dot(h16, a_s.astype(jnp.bfloat16),
                           preferred_element_type=jnp.float32)
    ald_ref[...] = jnp.dot(h16, a_d.astype(jnp.bfloat16),
                           preferred_element_type=jnp.float32)


def _dense1_body(x_ref, w_ref, as_ref, ad_ref, hs_ref, als_ref, ald_ref):
    _dense_common(x_ref[...], w_ref[...], as_ref[...], ad_ref[...],
                  hs_ref, als_ref, ald_ref)


def _dense2_body(o_ref, w_ref, as_ref, ad_ref, hs_ref, als_ref, ald_ref):
    t = jnp.maximum(jnp.concatenate([o_ref[0], o_ref[1]], axis=-1), 0.0)
    _dense_common(t, w_ref[...], as_ref[...], ad_ref[...],
                  hs_ref, als_ref, ald_ref)


_dense_out = [
    jax.ShapeDtypeStruct((NC, N, DH), jnp.float32),
    jax.ShapeDtypeStruct((N,), jnp.float32),
    jax.ShapeDtypeStruct((N,), jnp.float32),
]

_dense1 = pl.pallas_call(_dense1_body, out_shape=_dense_out)
_dense2 = pl.pallas_call(_dense2_body, out_shape=_dense_out)


_mesh = plsc.VectorSubcoreMesh(core_axis_name="c", subcore_axis_name="s")
_sc_params = pltpu.CompilerParams(needs_layout_passes=False,
                                  use_tc_tiling_on_sc=False)


@functools.partial(
    pl.kernel,
    out_type=jax.ShapeDtypeStruct((NC, N, DH), jnp.float32),
    mesh=_mesh,
    compiler_params=_sc_params,
    scratch_types=[
        pltpu.VMEM((N,), jnp.float32),
        pltpu.VMEM((N,), jnp.float32),
        pltpu.VMEM((EDGES_TILE,), jnp.int32),
        pltpu.VMEM((ROWS, CHUNK), jnp.int32),
        pltpu.VMEM((CHUNK, DH), jnp.float32),
        pltpu.VMEM((CHUNK, DH), jnp.float32),
        pltpu.VMEM((CHUNK,), jnp.float32),
        pltpu.VMEM((CHUNK,), jnp.float32),
        pltpu.VMEM((640,), jnp.float32),
        pltpu.VMEM_SHARED((N,), jnp.float32),
        pltpu.VMEM_SHARED((N, DH), jnp.float32),
        pltpu.SemaphoreType.DMA,
        pltpu.SemaphoreType.DMA,
        pltpu.SemaphoreType.DMA,
        pltpu.SemaphoreType.DMA,
        pltpu.SemaphoreType.DMA,
        pltpu.SemaphoreType.DMA,
    ],
)
def _edge_kernel(as_hbm, ad_hbm, h_hbm, src_hbm, dst_p1_hbm, zeros_hbm,
                 out_hbm, asv, adv, srcv, dstv2, rowsa, rowsb, pa, pb,
                 dbuf, den_sh, out_sh, ga, gb, sa, sb, pasem, pbsem):
    cid = lax.axis_index("c")
    sid = lax.axis_index("s")
    zero16 = jnp.zeros((L,), jnp.float32)

    def _zrow(r, carry):
        for q in range(QS):
            rowsa[r, pl.ds(q * L, L)] = zero16
        return carry
    lax.fori_loop(0, CHUNK, _zrow, 0)

    base = sid * OUT_ROWS_TILE
    nfull = OUT_ROWS_TILE // CHUNK
    rem = OUT_ROWS_TILE - nfull * CHUNK
    for k in range(nfull):
        pltpu.sync_copy(rowsa, out_sh.at[pl.ds(base + k * CHUNK, CHUNK)])
    pltpu.sync_copy(rowsa.at[pl.ds(0, rem)],
                    out_sh.at[pl.ds(base + nfull * CHUNK, rem)])

    @pl.when(sid == 0)
    def _():
        pltpu.sync_copy(rowsa.at[pl.ds(0, OUT_ROWS_REM)],
                        out_sh.at[pl.ds(OUT_REM_BASE, OUT_ROWS_REM)])
        pltpu.sync_copy(zeros_hbm, den_sh)

    plsc.subcore_barrier()

    pltpu.sync_copy(as_hbm, asv)
    pltpu.sync_copy(ad_hbm, adv)
    pltpu.sync_copy(src_hbm.at[pl.ds(sid * EDGES_TILE, EDGES_TILE)], srcv)
    pltpu.sync_copy(dst_p1_hbm.at[sid], dstv2)

    def _issue_gather(r, buf, sem):
        pltpu.async_copy(
            h_hbm.at[cid].at[srcv.at[pl.ds(r * CHUNK, CHUNK)]], buf, sem)

    def _drain_rows(buf, sem):
        pltpu.make_async_copy(h_hbm.at[0].at[pl.ds(0, CHUNK)], buf, sem).wait()

    def _p_scale(r, rows, pv):
        for g in range(CHUNK // L):
            s16 = srcv[pl.ds(r * CHUNK + g * L, L)]
            d16 = dstv2[r, pl.ds(g * L, L)]
            a1 = plsc.load_gather(asv, [s16])
            a2 = plsc.load_gather(adv, [d16])
            e = a1 + a2
            w = jnp.where(e >= 0.0, e, e * _NEG_SLOPE)
            pv[pl.ds(g * L, L)] = jnp.exp(w)

        def _scale(jj, c2):
            j0 = jj * 8
            for u in range(8):
                j = j0 + u
                c16 = plsc.load_gather(pv, [jnp.full((L,), j, jnp.int32)])
                for q in range(QS):
                    rows[j, pl.ds(q * L, L)] = rows[j, pl.ds(q * L, L)] * c16
            return c2
        lax.fori_loop(0, CHUNK // 8, _scale, 0)

    _issue_gather(0, rowsa, ga)
    _issue_gather(1, rowsb, gb)

    def _scan(k, carry):
        r0 = 2 * k
        r1 = 2 * k + 1
        _drain_rows(rowsa, ga)
        _p_scale(r0, rowsa, pa)
        pltpu.async_copy(rowsa, out_sh.at[dstv2.at[r0]], sa, add=True)
        pltpu.async_copy(pa, den_sh.at[dstv2.at[r0]], pasem, add=True)
        _drain_rows(rowsb, gb)
        _p_scale(r1, rowsb, pb)
        pltpu.async_copy(rowsb, out_sh.at[dstv2.at[r1]], sb, add=True)
        pltpu.async_copy(pb, den_sh.at[dstv2.at[r1]], pbsem, add=True)
        _drain_rows(rowsa, sa)
        pltpu.make_async_copy(as_hbm.at[pl.ds(0, CHUNK)], pa, pasem).wait()
        _issue_gather(jnp.minimum(r0 + 2, ROWS - 1), rowsa, ga)
        _drain_rows(rowsb, sb)
        pltpu.make_async_copy(as_hbm.at[pl.ds(0, CHUNK)], pb, pbsem).wait()
        _issue_gather(jnp.minimum(r1 + 2, ROWS - 1), rowsb, gb)
        return carry
    lax.fori_loop(0, ROWS // 2, _scan, 0)

    _drain_rows(rowsa, ga)
    _drain_rows(rowsb, gb)

    plsc.subcore_barrier()

    pltpu.sync_copy(den_sh.at[pl.ds(base, OUT_ROWS_TILE)],
                    dbuf.at[pl.ds(0, OUT_ROWS_TILE)])

    def _div_rows(rows, nrows, dbase):
        def _dv(jj, c2):
            j0 = jj * 4
            for u in range(4):
                j = j0 + u
                d16 = plsc.load_gather(dbuf, [jnp.full((L,), dbase + j,
                                                       jnp.int32)])
                rec = 1.0 / (d16 + 1e-16)
                for q in range(QS):
                    rows[j, pl.ds(q * L, L)] = rows[j, pl.ds(q * L, L)] * rec
            return c2
        lax.fori_loop(0, nrows // 4, _dv, 0)

    for k in range(nfull + 1):
        size = CHUNK if k < nfull else rem
        off = base + k * CHUNK
        pltpu.sync_copy(out_sh.at[pl.ds(off, size)], rowsa.at[pl.ds(0, size)])
        _div_rows(rowsa, size, k * CHUNK)
        pltpu.sync_copy(rowsa.at[pl.ds(0, size)],
                        out_hbm.at[cid].at[pl.ds(off, size)])

    @pl.when(sid == 0)
    def _():
        pltpu.sync_copy(den_sh.at[pl.ds(OUT_REM_BASE, OUT_ROWS_REM)],
                        dbuf.at[pl.ds(0, OUT_ROWS_REM)])
        pltpu.sync_copy(out_sh.at[pl.ds(OUT_REM_BASE, OUT_ROWS_REM)],
                        rowsa.at[pl.ds(0, OUT_ROWS_REM)])
        _div_rows(rowsa, OUT_ROWS_REM, 0)
        pltpu.sync_copy(rowsa.at[pl.ds(0, OUT_ROWS_REM)],
                        out_hbm.at[cid].at[pl.ds(OUT_REM_BASE, OUT_ROWS_REM)])


@functools.partial(
    pl.kernel,
    out_type=jax.ShapeDtypeStruct((C,), jnp.float32),
    mesh=_mesh,
    compiler_params=_sc_params,
    scratch_types=[
        pltpu.VMEM((C,), jnp.int32),
        pltpu.VMEM((C, DH), jnp.float32),
        pltpu.VMEM((C, DH), jnp.float32),
        pltpu.VMEM((D,), jnp.float32),
        pltpu.VMEM((C,), jnp.float32),
    ],
)
def _score_kernel(o_hbm, cand_hbm, w_hbm, out_hbm, candv, r0, r1, wv, sv):
    cid = lax.axis_index("c")
    sid = lax.axis_index("s")

    @pl.when((cid == 0) & (sid == 0))
    def _():
        pltpu.sync_copy(cand_hbm, candv)
        pltpu.sync_copy(w_hbm, wv)
        pltpu.sync_copy(o_hbm.at[0].at[candv], r0)
        pltpu.sync_copy(o_hbm.at[1].at[candv], r1)

        def _b16(v):
            u = plsc.bitcast(v, jnp.int32)
            r = u + 0x7FFF + ((u >> 16) & 1)
            return plsc.bitcast(r & jnp.int32(-65536), jnp.float32)

        for g in range(C // L):
            rid = lax.iota(jnp.int32, L) + g * L

            def _f(f, acc):
                f16 = jnp.full((L,), f, jnp.int32)
                c0 = _b16(jnp.maximum(plsc.load_gather(r0, [rid, f16]), 0.0))
                c1 = _b16(jnp.maximum(plsc.load_gather(r1, [rid, f16]), 0.0))
                w0 = _b16(plsc.load_gather(wv, [f16]))
                w1 = _b16(plsc.load_gather(wv, [f16 + DH]))
                return acc + c0 * w0 + c1 * w1
            acc = lax.fori_loop(0, DH, _f, jnp.zeros((L,), jnp.float32))
            sv[pl.ds(g * L, L)] = acc
        pltpu.sync_copy(sv, out_hbm)


def kernel(x, edge_index, cand_idxs, W1, a_src1, a_dst1,
           W2, a_src2, a_dst2, w_score, b_score):
    src = edge_index[0]
    dst_p1 = edge_index[1].reshape(NS, ROWS, CHUNK)

    zeros_n = jnp.zeros((N,), jnp.float32)
    hs1, as1, ad1 = _dense1(x, W1, a_src1, a_dst1)
    out1 = _edge_kernel(as1, ad1, hs1, src, dst_p1, zeros_n)
    hs2, as2, ad2 = _dense2(out1, W2, a_src2, a_dst2)
    out2 = _edge_kernel(as2, ad2, hs2, src, dst_p1, zeros_n)
    scores = _score_kernel(out2, cand_idxs, w_score)
    return scores + b_score


# speedup vs baseline: 2.1688x; 1.3157x over previous
import functools

import jax
import jax.numpy as jnp
from jax import lax
from jax.experimental import pallas as pl
from jax.experimental.pallas import tpu as pltpu
from jax.experimental.pallas import tpu_sc as plsc

N = 10000
E = 320000
D = 128
DH = D // 2
C = 32

L = 16
NC = 2
NS = 16
CHUNK = 80

EDGES_TILE = E // NS
ROWS = EDGES_TILE // CHUNK
QS = DH // L
N_DEN = N
N_OUT = N
OUT_ROWS_TILE = 624
OUT_ROWS_REM = N - NS * OUT_ROWS_TILE
OUT_REM_BASE = NS * OUT_ROWS_TILE
OUT_ZERO_REM = N_OUT - NS * OUT_ROWS_TILE

_NEG_SLOPE = 0.2


def _dense_common(x, W, a_s, a_d, hs_ref, als_ref, ald_ref):
    h = jnp.dot(x.astype(jnp.bfloat16), W.astype(jnp.bfloat16),
                preferred_element_type=jnp.float32)
    hs_ref[0] = h[:, :DH]
    hs_ref[1] = h[:, DH:]
    h16 = h.astype(jnp.bfloat16)
    als_ref[...] = jnp.dot(h16, a_s.astype(jnp.bfloat16),
                           preferred_element_type=jnp.float32)
    ald_ref[...] = jnp.dot(h16, a_d.astype(jnp.bfloat16),
                           preferred_element_type=jnp.float32)


def _dense1_body(x_ref, w_ref, as_ref, ad_ref, hs_ref, als_ref, ald_ref):
    _dense_common(x_ref[...], w_ref[...], as_ref[...], ad_ref[...],
                  hs_ref, als_ref, ald_ref)


def _dense2_body(o_ref, w_ref, as_ref, ad_ref, hs_ref, als_ref, ald_ref):
    t = jnp.maximum(jnp.concatenate([o_ref[0], o_ref[1]], axis=-1), 0.0)
    _dense_common(t, w_ref[...], as_ref[...], ad_ref[...],
                  hs_ref, als_ref, ald_ref)


_dense_out = [
    jax.ShapeDtypeStruct((NC, N, DH), jnp.float32),
    jax.ShapeDtypeStruct((N,), jnp.float32),
    jax.ShapeDtypeStruct((N,), jnp.float32),
]

_dense1 = pl.pallas_call(_dense1_body, out_shape=_dense_out)
_dense2 = pl.pallas_call(_dense2_body, out_shape=_dense_out)


_mesh = plsc.VectorSubcoreMesh(core_axis_name="c", subcore_axis_name="s")
_sc_params = pltpu.CompilerParams(needs_layout_passes=False,
                                  use_tc_tiling_on_sc=False)


@functools.partial(
    pl.kernel,
    out_type=jax.ShapeDtypeStruct((NC, N, DH), jnp.float32),
    mesh=_mesh,
    compiler_params=_sc_params,
    scratch_types=[
        pltpu.VMEM((N,), jnp.float32),
        pltpu.VMEM((N,), jnp.float32),
        pltpu.VMEM((EDGES_TILE,), jnp.int32),
        pltpu.VMEM((ROWS, CHUNK), jnp.int32),
        pltpu.VMEM((CHUNK, DH), jnp.float32),
        pltpu.VMEM((CHUNK, DH), jnp.float32),
        pltpu.VMEM((CHUNK, DH), jnp.float32),
        pltpu.VMEM((CHUNK, DH), jnp.float32),
        pltpu.VMEM((CHUNK,), jnp.float32),
        pltpu.VMEM((CHUNK,), jnp.float32),
        pltpu.VMEM((CHUNK,), jnp.float32),
        pltpu.VMEM((CHUNK,), jnp.float32),
        pltpu.VMEM((640,), jnp.float32),
        pltpu.VMEM_SHARED((N,), jnp.float32),
        pltpu.VMEM_SHARED((N, DH), jnp.float32),
    ] + [pltpu.SemaphoreType.DMA] * 12,
)
def _edge_kernel(as_hbm, ad_hbm, h_hbm, src_hbm, dst_p1_hbm, zeros_hbm,
                 out_hbm, asv, adv, srcv, dstv2, rowsa, rowsb, rowsc, rowsd,
                 pa, pb, pc, pd, dbuf, den_sh, out_sh,
                 ga, gb, gc, gd, sa, sb, sc_, sd, pas, pbs, pcs, pds):
    cid = lax.axis_index("c")
    sid = lax.axis_index("s")
    zero16 = jnp.zeros((L,), jnp.float32)

    def _zrow(r, carry):
        for q in range(QS):
            rowsa[r, pl.ds(q * L, L)] = zero16
        return carry
    lax.fori_loop(0, CHUNK, _zrow, 0)

    base = sid * OUT_ROWS_TILE
    nfull = OUT_ROWS_TILE // CHUNK
    rem = OUT_ROWS_TILE - nfull * CHUNK
    for k in range(nfull):
        pltpu.sync_copy(rowsa, out_sh.at[pl.ds(base + k * CHUNK, CHUNK)])
    pltpu.sync_copy(rowsa.at[pl.ds(0, rem)],
                    out_sh.at[pl.ds(base + nfull * CHUNK, rem)])

    @pl.when(sid == 0)
    def _():
        pltpu.sync_copy(rowsa.at[pl.ds(0, OUT_ROWS_REM)],
                        out_sh.at[pl.ds(OUT_REM_BASE, OUT_ROWS_REM)])
        pltpu.sync_copy(zeros_hbm, den_sh)

    plsc.subcore_barrier()

    pltpu.sync_copy(as_hbm, asv)
    pltpu.sync_copy(ad_hbm, adv)
    pltpu.sync_copy(src_hbm.at[pl.ds(sid * EDGES_TILE, EDGES_TILE)], srcv)
    pltpu.sync_copy(dst_p1_hbm.at[sid], dstv2)

    def _issue_gather(r, buf, sem):
        pltpu.async_copy(
            h_hbm.at[cid].at[srcv.at[pl.ds(r * CHUNK, CHUNK)]], buf, sem)

    def _drain_rows(buf, sem):
        pltpu.make_async_copy(h_hbm.at[0].at[pl.ds(0, CHUNK)], buf, sem).wait()

    def _p_scale(r, rows, pv):
        for g in range(CHUNK // L):
            s16 = srcv[pl.ds(r * CHUNK + g * L, L)]
            d16 = dstv2[r, pl.ds(g * L, L)]
            a1 = plsc.load_gather(asv, [s16])
            a2 = plsc.load_gather(adv, [d16])
            e = a1 + a2
            w = jnp.where(e >= 0.0, e, e * _NEG_SLOPE)
            pv[pl.ds(g * L, L)] = jnp.exp(w)

        def _scale(jj, c2):
            j0 = jj * 8
            for u in range(8):
                j = j0 + u
                c16 = plsc.load_gather(pv, [jnp.full((L,), j, jnp.int32)])
                for q in range(QS):
                    rows[j, pl.ds(q * L, L)] = rows[j, pl.ds(q * L, L)] * c16
            return c2
        lax.fori_loop(0, CHUNK // 8, _scale, 0)

    BUFS = [(rowsa, pa, ga, sa, pas), (rowsb, pb, gb, sb, pbs),
            (rowsc, pc, gc, sc_, pcs), (rowsd, pd, gd, sd, pds)]

    def _drain_p(pv, sem):
        pltpu.make_async_copy(as_hbm.at[pl.ds(0, CHUNK)], pv, sem).wait()

    def _step(r, buf):
        rows, pv, gsem, ssem, psem = buf
        _drain_rows(rows, gsem)
        _p_scale(r, rows, pv)
        pltpu.async_copy(rows, out_sh.at[dstv2.at[r]], ssem, add=True)
        pltpu.async_copy(pv, den_sh.at[dstv2.at[r]], psem, add=True)

    def _reissue(r, buf):
        rows, pv, gsem, ssem, psem = buf
        _drain_rows(rows, ssem)
        _drain_p(pv, psem)
        _issue_gather(jnp.minimum(r + 4, ROWS - 1), rows, gsem)

    for i, buf in enumerate(BUFS):
        _issue_gather(i, buf[0], buf[2])

    NBODY = (ROWS - 2) // 4

    def _scan(k, carry):
        rb = 4 * k
        _step(rb + 0, BUFS[0])
        _step(rb + 1, BUFS[1])
        _reissue(rb + 0, BUFS[0])
        _step(rb + 2, BUFS[2])
        _reissue(rb + 1, BUFS[1])
        _step(rb + 3, BUFS[3])
        _reissue(rb + 2, BUFS[2])
        _reissue(rb + 3, BUFS[3])
        return carry
    lax.fori_loop(0, NBODY, _scan, 0)

    _step(ROWS - 2, BUFS[0])
    _step(ROWS - 1, BUFS[1])
    _drain_rows(rowsc, gc)
    _drain_rows(rowsd, gd)
    _drain_rows(rowsa, sa)
    _drain_p(pa, pas)
    _drain_rows(rowsb, sb)
    _drain_p(pb, pbs)

    plsc.subcore_barrier()

    pltpu.sync_copy(den_sh.at[pl.ds(base, OUT_ROWS_TILE)],
                    dbuf.at[pl.ds(0, OUT_ROWS_TILE)])

    def _div_rows(rows, nrows, dbase):
        def _dv(jj, c2):
            j0 = jj * 4
            for u in range(4):
                j = j0 + u
                d16 = plsc.load_gather(dbuf, [jnp.full((L,), dbase + j,
                                                       jnp.int32)])
                rec = 1.0 / (d16 + 1e-16)
                for q in range(QS):
                    rows[j, pl.ds(q * L, L)] = rows[j, pl.ds(q * L, L)] * rec
            return c2
        lax.fori_loop(0, nrows // 4, _dv, 0)

    for k in range(nfull + 1):
        size = CHUNK if k < nfull else rem
        off = base + k * CHUNK
        pltpu.sync_copy(out_sh.at[pl.ds(off, size)], rowsa.at[pl.ds(0, size)])
        _div_rows(rowsa, size, k * CHUNK)
        pltpu.sync_copy(rowsa.at[pl.ds(0, size)],
                        out_hbm.at[cid].at[pl.ds(off, size)])

    @pl.when(sid == 0)
    def _():
        pltpu.sync_copy(den_sh.at[pl.ds(OUT_REM_BASE, OUT_ROWS_REM)],
                        dbuf.at[pl.ds(0, OUT_ROWS_REM)])
        pltpu.sync_copy(out_sh.at[pl.ds(OUT_REM_BASE, OUT_ROWS_REM)],
                        rowsa.at[pl.ds(0, OUT_ROWS_REM)])
        _div_rows(rowsa, OUT_ROWS_REM, 0)
        pltpu.sync_copy(rowsa.at[pl.ds(0, OUT_ROWS_REM)],
                        out_hbm.at[cid].at[pl.ds(OUT_REM_BASE, OUT_ROWS_REM)])


@functools.partial(
    pl.kernel,
    out_type=jax.ShapeDtypeStruct((C,), jnp.float32),
    mesh=_mesh,
    compiler_params=_sc_params,
    scratch_types=[
        pltpu.VMEM((C,), jnp.int32),
        pltpu.VMEM((C, DH), jnp.float32),
        pltpu.VMEM((C, DH), jnp.float32),
        pltpu.VMEM((D,), jnp.float32),
        pltpu.VMEM((C,), jnp.float32),
    ],
)
def _score_kernel(o_hbm, cand_hbm, w_hbm, out_hbm, candv, r0, r1, wv, sv):
    cid = lax.axis_index("c")
    sid = lax.axis_index("s")

    @pl.when((cid == 0) & (sid == 0))
    def _():
        pltpu.sync_copy(cand_hbm, candv)
        pltpu.sync_copy(w_hbm, wv)
        pltpu.sync_copy(o_hbm.at[0].at[candv], r0)
        pltpu.sync_copy(o_hbm.at[1].at[candv], r1)

        def _b16(v):
            u = plsc.bitcast(v, jnp.int32)
            r = u + 0x7FFF + ((u >> 16) & 1)
            return plsc.bitcast(r & jnp.int32(-65536), jnp.float32)

        for g in range(C // L):
            rid = lax.iota(jnp.int32, L) + g * L

            def _f(f, acc):
                f16 = jnp.full((L,), f, jnp.int32)
                c0 = _b16(jnp.maximum(plsc.load_gather(r0, [rid, f16]), 0.0))
                c1 = _b16(jnp.maximum(plsc.load_gather(r1, [rid, f16]), 0.0))
                w0 = _b16(plsc.load_gather(wv, [f16]))
                w1 = _b16(plsc.load_gather(wv, [f16 + DH]))
                return acc + c0 * w0 + c1 * w1
            acc = lax.fori_loop(0, DH, _f, jnp.zeros((L,), jnp.float32))
            sv[pl.ds(g * L, L)] = acc
        pltpu.sync_copy(sv, out_hbm)


def kernel(x, edge_index, cand_idxs, W1, a_src1, a_dst1,
           W2, a_src2, a_dst2, w_score, b_score):
    src = edge_index[0]
    dst_p1 = edge_index[1].reshape(NS, ROWS, CHUNK)

    zeros_n = jnp.zeros((N,), jnp.float32)
    hs1, as1, ad1 = _dense1(x, W1, a_src1, a_dst1)
    out1 = _edge_kernel(as1, ad1, hs1, src, dst_p1, zeros_n)
    hs2, as2, ad2 = _dense2(out1, W2, a_src2, a_dst2)
    out2 = _edge_kernel(as2, ad2, hs2, src, dst_p1, zeros_n)
    scores = _score_kernel(out2, cand_idxs, w_score)
    return scores + b_score
